# baseline (speedup 1.0000x reference)
import jax
import jax.numpy as jnp
from jax import lax
from jax.experimental import pallas as pl
from jax.experimental.pallas import tpu as pltpu
from jax.experimental.pallas import tpu_sc as plsc

N_NODES = 10000
N_EDGES = 320000
D = 128

NC = 2
NS = 16
NW = NC * NS
SCAT = 128
BUF = SCAT
NBLK = N_EDGES // BUF
NBUF = 3
N_PAD = 10112
ROWS_PER_TILE = N_PAD // NS
ZROWS = 128


def _sc_scatter_body(hu_hbm, v_hbm, out_hbm,
                     idx0, idx1, idx2, rows0, rows1, rows2,
                     sem_i0, sem_i1, sem_i2, sem_r0, sem_r1, sem_r2,
                     sem_s0, sem_s1, sem_s2,
                     gamma_sh):
  cid = lax.axis_index("c")
  sid = lax.axis_index("s")
  wid = sid * NC + cid
  idx_v = (idx0, idx1, idx2)
  rows_v = (rows0, rows1, rows2)
  sem_i = (sem_i0, sem_i1, sem_i2)
  sem_r = (sem_r0, sem_r1, sem_r2)
  sem_s = (sem_s0, sem_s1, sem_s2)

  start = wid * NBLK // NW
  end = (wid + 1) * NBLK // NW

  def issue_loads(b, blk):
    pltpu.async_copy(v_hbm.at[pl.ds(blk * BUF, BUF)], idx_v[b], sem_i[b])
    pltpu.async_copy(hu_hbm.at[pl.ds(blk * BUF, BUF)], rows_v[b], sem_r[b])

  def wait_loads(b, blk):
    pltpu.make_async_copy(v_hbm.at[pl.ds(blk * BUF, BUF)], idx_v[b],
                          sem_i[b]).wait()
    pltpu.make_async_copy(hu_hbm.at[pl.ds(blk * BUF, BUF)], rows_v[b],
                          sem_r[b]).wait()

  for b in range(1, NBUF):
    @pl.when(start + b < end)
    def _():
      issue_loads(b, start + b)

  zero16 = jnp.zeros((16,), jnp.float32)

  def zrow(i, carry):
    for j in range(D // 16):
      rows0[i, pl.ds(j * 16, 16)] = zero16
    return carry

  lax.fori_loop(0, ZROWS, zrow, 0)
  zbase = sid * ROWS_PER_TILE
  for k in range(ROWS_PER_TILE // ZROWS):
    pltpu.sync_copy(rows0.at[pl.ds(0, ZROWS)],
                    gamma_sh.at[pl.ds(zbase + k * ZROWS, ZROWS)])
  rem = ROWS_PER_TILE % ZROWS
  if rem:
    pltpu.sync_copy(
        rows0.at[pl.ds(0, rem)],
        gamma_sh.at[pl.ds(zbase + (ROWS_PER_TILE // ZROWS) * ZROWS, rem)])

  @pl.when(start < end)
  def _():
    issue_loads(0, start)

  plsc.subcore_barrier()


  def issue_scatter(b):
    pltpu.async_copy(rows_v[b], gamma_sh.at[idx_v[b]], sem_s[b], add=True)

  def wait_scatter(b):
    pltpu.make_async_copy(rows_v[b], gamma_sh.at[idx_v[b]], sem_s[b]).wait()

  def tri_body(g, carry):
    base = start + NBUF * g
    for b in range(NBUF):
      blk = base + b

      @pl.when(blk < end)
      def _():
        wait_loads(b, blk)
        issue_scatter(b)
        pb = (b - 1) % NBUF

        @pl.when(blk - 1 >= start)
        def _():
          wait_scatter(pb)

          @pl.when(blk - 1 + NBUF < end)
          def _():
            issue_loads(pb, blk - 1 + NBUF)
    return carry

  max_trips = (NBLK + NW - 1) // NW
  lax.fori_loop(0, (max_trips + NBUF - 1) // NBUF, tri_body, 0)

  for b in range(NBUF):
    @pl.when((end > start) & ((end - 1 - start) % NBUF == b))
    def _():
      wait_scatter(b)

  plsc.subcore_barrier()

  pltpu.sync_copy(
      gamma_sh.at[pl.ds(sid * ROWS_PER_TILE, ROWS_PER_TILE)],
      out_hbm.at[cid, pl.ds(sid * ROWS_PER_TILE, ROWS_PER_TILE)],
  )


def _sc_scatter(h_u, v):
  mesh = plsc.VectorSubcoreMesh(
      core_axis_name="c", subcore_axis_name="s", num_cores=NC, num_subcores=NS)
  return pl.kernel(
      _sc_scatter_body,
      out_type=jax.ShapeDtypeStruct((NC, N_PAD, D), jnp.float32),
      mesh=mesh,
      scratch_types=[
          pltpu.VMEM((SCAT,), jnp.int32),
          pltpu.VMEM((SCAT,), jnp.int32),
          pltpu.VMEM((SCAT,), jnp.int32),
          pltpu.VMEM((BUF, D), jnp.float32),
          pltpu.VMEM((BUF, D), jnp.float32),
          pltpu.VMEM((BUF, D), jnp.float32),
          pltpu.SemaphoreType.DMA,
          pltpu.SemaphoreType.DMA,
          pltpu.SemaphoreType.DMA,
          pltpu.SemaphoreType.DMA,
          pltpu.SemaphoreType.DMA,
          pltpu.SemaphoreType.DMA,
          pltpu.SemaphoreType.DMA,
          pltpu.SemaphoreType.DMA,
          pltpu.SemaphoreType.DMA,
          pltpu.VMEM_SHARED((N_PAD, D), jnp.float32),
      ],
  )(h_u, v)


ROW_BLK = 2000


def _tc_apply_body(gp_ref, w_ref, b_ref, o_ref):
  g = gp_ref[0] + gp_ref[1]
  acc = lax.dot_general(g, w_ref[...], (((1,), (1,)), ((), ())),
                        preferred_element_type=jnp.float32)
  act = acc + b_ref[...]
  o_ref[...] = jnp.where(act >= 0, act, 0.01 * act)


def _tc_apply(partial, w, b):
  return pl.pallas_call(
      _tc_apply_body,
      grid=(N_NODES // ROW_BLK,),
      in_specs=[
          pl.BlockSpec((NC, ROW_BLK, D), lambda i: (0, i, 0)),
          pl.BlockSpec((D, D), lambda i: (0, 0)),
          pl.BlockSpec((1, D), lambda i: (0, 0)),
      ],
      out_specs=pl.BlockSpec((ROW_BLK, D), lambda i: (i, 0)),
      out_shape=jax.ShapeDtypeStruct((N_NODES, D), jnp.float32),
  )(partial, w, b.reshape(1, D))


def kernel(H_v, H_u, X_e, batch_indices, V, U, X_v, W, b):
  partial = _sc_scatter(H_u, V)
  return _tc_apply(partial, W, b)


# speedup vs baseline: 1.1354x; 1.1354x over previous
import jax
import jax.numpy as jnp
from jax import lax
from jax.experimental import pallas as pl
from jax.experimental.pallas import tpu as pltpu
from jax.experimental.pallas import tpu_sc as plsc

N_NODES = 10000
N_EDGES = 320000
D = 128

NC = 2
NS = 16
NW = NC * NS
SCAT = 128
BUF = SCAT
NBLK = N_EDGES // BUF
NBUF = 3
N_PAD = 10112
ROWS_PER_TILE = N_PAD // NS
ZROWS = 128


def _sc_scatter_body(hu_hbm, v_hbm, out_hbm,
                     idx0, idx1, idx2, rows0, rows1, rows2,
                     sem_i0, sem_i1, sem_i2, sem_r0, sem_r1, sem_r2,
                     gamma_sh):
  cid = lax.axis_index("c")
  sid = lax.axis_index("s")
  wid = sid * NC + cid
  idx_v = (idx0, idx1, idx2)
  rows_v = (rows0, rows1, rows2)
  sem_i = (sem_i0, sem_i1, sem_i2)
  sem_r = (sem_r0, sem_r1, sem_r2)

  start = wid * NBLK // NW
  end = (wid + 1) * NBLK // NW

  def issue_loads(b, blk):
    pltpu.async_copy(v_hbm.at[pl.ds(blk * BUF, BUF)], idx_v[b], sem_i[b])
    pltpu.async_copy(hu_hbm.at[pl.ds(blk * BUF, BUF)], rows_v[b], sem_r[b])

  def wait_loads(b, blk):
    pltpu.make_async_copy(v_hbm.at[pl.ds(blk * BUF, BUF)], idx_v[b],
                          sem_i[b]).wait()
    pltpu.make_async_copy(hu_hbm.at[pl.ds(blk * BUF, BUF)], rows_v[b],
                          sem_r[b]).wait()

  for b in range(1, NBUF):
    @pl.when(start + b < end)
    def _():
      issue_loads(b, start + b)

  zero16 = jnp.zeros((16,), jnp.float32)

  def zrow(i, carry):
    for j in range(D // 16):
      rows0[i, pl.ds(j * 16, 16)] = zero16
    return carry

  lax.fori_loop(0, ZROWS, zrow, 0)
  zbase = sid * ROWS_PER_TILE
  for k in range(ROWS_PER_TILE // ZROWS):
    pltpu.sync_copy(rows0.at[pl.ds(0, ZROWS)],
                    gamma_sh.at[pl.ds(zbase + k * ZROWS, ZROWS)])
  rem = ROWS_PER_TILE % ZROWS
  if rem:
    pltpu.sync_copy(
        rows0.at[pl.ds(0, rem)],
        gamma_sh.at[pl.ds(zbase + (ROWS_PER_TILE // ZROWS) * ZROWS, rem)])

  @pl.when(start < end)
  def _():
    issue_loads(0, start)

  plsc.subcore_barrier()


  def tri_body(g, carry):
    base = start + NBUF * g
    for b in range(NBUF):
      blk = base + b

      @pl.when(blk < end)
      def _():
        wait_loads(b, blk)
        pltpu.sync_copy(rows_v[b], gamma_sh.at[idx_v[b]], add=True)

        @pl.when(blk + NBUF < end)
        def _():
          issue_loads(b, blk + NBUF)
    return carry

  max_trips = (NBLK + NW - 1) // NW
  lax.fori_loop(0, (max_trips + NBUF - 1) // NBUF, tri_body, 0)
  plsc.subcore_barrier()

  pltpu.sync_copy(
      gamma_sh.at[pl.ds(sid * ROWS_PER_TILE, ROWS_PER_TILE)],
      out_hbm.at[cid, pl.ds(sid * ROWS_PER_TILE, ROWS_PER_TILE)],
  )


def _sc_scatter(h_u, v):
  mesh = plsc.VectorSubcoreMesh(
      core_axis_name="c", subcore_axis_name="s", num_cores=NC, num_subcores=NS)
  return pl.kernel(
      _sc_scatter_body,
      out_type=jax.ShapeDtypeStruct((NC, N_PAD, D), jnp.float32),
      mesh=mesh,
      scratch_types=[
          pltpu.VMEM((SCAT,), jnp.int32),
          pltpu.VMEM((SCAT,), jnp.int32),
          pltpu.VMEM((SCAT,), jnp.int32),
          pltpu.VMEM((BUF, D), jnp.float32),
          pltpu.VMEM((BUF, D), jnp.float32),
          pltpu.VMEM((BUF, D), jnp.float32),
          pltpu.SemaphoreType.DMA,
          pltpu.SemaphoreType.DMA,
          pltpu.SemaphoreType.DMA,
          pltpu.SemaphoreType.DMA,
          pltpu.SemaphoreType.DMA,
          pltpu.SemaphoreType.DMA,
          pltpu.VMEM_SHARED((N_PAD, D), jnp.float32),
      ],
  )(h_u, v)


ROW_BLK = 5000


def _tc_apply_body(gp_ref, w_ref, b_ref, o_ref):
  g = gp_ref[0] + gp_ref[1]
  acc = lax.dot_general(g, w_ref[...], (((1,), (1,)), ((), ())),
                        preferred_element_type=jnp.float32)
  act = acc + b_ref[...]
  o_ref[...] = jnp.where(act >= 0, act, 0.01 * act)


def _tc_apply(partial, w, b):
  return pl.pallas_call(
      _tc_apply_body,
      grid=(N_NODES // ROW_BLK,),
      in_specs=[
          pl.BlockSpec((NC, ROW_BLK, D), lambda i: (0, i, 0)),
          pl.BlockSpec((D, D), lambda i: (0, 0)),
          pl.BlockSpec((1, D), lambda i: (0, 0)),
      ],
      out_specs=pl.BlockSpec((ROW_BLK, D), lambda i: (i, 0)),
      out_shape=jax.ShapeDtypeStruct((N_NODES, D), jnp.float32),
  )(partial, w, b.reshape(1, D))


def kernel(H_v, H_u, X_e, batch_indices, V, U, X_v, W, b):
  partial = _sc_scatter(H_u, V)
  return _tc_apply(partial, W, b)
